# Initial kernel scaffold; baseline (speedup 1.0000x reference)
import functools

import jax
import jax.numpy as jnp
from jax import lax
from jax.experimental import pallas as pl
from jax.experimental.pallas import tpu as pltpu
from jax.experimental.pallas import tpu_sc as plsc

N = 10000
E = 320000
F = 128
EPS = 1e-5

NC, NS = 2, 16
NW = NC * NS
EPW = E // NW
CH = 80
NCH = EPW // CH
RPT = N // NS
CNTW = 8

_MESH = plsc.VectorSubcoreMesh(
    core_axis_name="c", subcore_axis_name="s", num_cores=NC, num_subcores=NS)


def _make_sc_agg(with_counts: bool):
  out_type = [jax.ShapeDtypeStruct((NC, N, F), jnp.float32)]
  scratch = [
      pltpu.VMEM_SHARED((N, F), jnp.float32),
      pltpu.VMEM((NCH, CH), jnp.int32),
      pltpu.VMEM((NCH, CH), jnp.int32),
      pltpu.VMEM((CH, F), jnp.float32),
      pltpu.SemaphoreType.DMA,
  ]
  if with_counts:
    out_type.append(jax.ShapeDtypeStruct((NC, N, CNTW), jnp.float32))
    scratch += [
        pltpu.VMEM_SHARED((N, CNTW), jnp.float32),
        pltpu.VMEM((CH, CNTW), jnp.float32),
    ]

  @functools.partial(pl.kernel, out_type=out_type, mesh=_MESH,
                     scratch_types=scratch)
  def sc_agg(h_hbm, src_hbm, dst_hbm, ones_hbm, zf_hbm, zc_hbm,
             out_hbm, *rest):
    if with_counts:
      cnt_out_hbm, agg_sh, src_v, dst_v, rows_v, sem, cnt_sh, ones_v = rest
    else:
      agg_sh, src_v, dst_v, rows_v, sem = rest
    cid = lax.axis_index("c")
    sid = lax.axis_index("s")
    wid = sid * NC + cid

    pltpu.sync_copy(zf_hbm, agg_sh.at[pl.ds(sid * RPT, RPT)])
    pltpu.sync_copy(src_hbm.at[pl.ds(wid * NCH, NCH)], src_v)
    pltpu.sync_copy(dst_hbm.at[pl.ds(wid * NCH, NCH)], dst_v)
    if with_counts:
      pltpu.sync_copy(zc_hbm, cnt_sh.at[pl.ds(sid * RPT, RPT)])
      pltpu.sync_copy(ones_hbm, ones_v)
    plsc.subcore_barrier()

    def chunk(j, carry):
      pltpu.async_copy(h_hbm.at[src_v.at[j]], rows_v, sem).wait()
      pltpu.sync_copy(rows_v, agg_sh.at[dst_v.at[j]], add=True)
      if with_counts:
        pltpu.sync_copy(ones_v, cnt_sh.at[dst_v.at[j]], add=True)
      return carry

    lax.fori_loop(0, NCH, chunk, 0)
    plsc.subcore_barrier()

    pltpu.sync_copy(agg_sh.at[pl.ds(sid * RPT, RPT)],
                    out_hbm.at[cid, pl.ds(sid * RPT, RPT)])
    if with_counts:
      pltpu.sync_copy(cnt_sh.at[pl.ds(sid * RPT, RPT)],
                      cnt_out_hbm.at[cid, pl.ds(sid * RPT, RPT)])

  return sc_agg


_sc_agg_counts = _make_sc_agg(True)
_sc_agg = _make_sc_agg(False)


def _dot(a, b):
  return lax.dot_general(a, b, (((1,), (0,)), ((), ())),
                         precision=lax.Precision.HIGHEST,
                         preferred_element_type=jnp.float32)


def _tc_layer_body(p_ref, c_ref, h_ref, wl_ref, wr_ref, b_ref, g_ref,
                   be_ref, o_ref, *, with_bn):
  cnt = jnp.maximum(c_ref[0, :, 0:1] + c_ref[1, :, 0:1], 1.0)
  mean = (p_ref[0] + p_ref[1]) / cnt
  y = _dot(mean, wl_ref[...]) + _dot(h_ref[...], wr_ref[...]) + b_ref[...]
  if with_bn:
    mu = jnp.mean(y, axis=0, keepdims=True)
    var = jnp.mean((y - mu) * (y - mu), axis=0, keepdims=True)
    y = (y - mu) * lax.rsqrt(var + EPS) * g_ref[...] + be_ref[...]
    y = jnp.maximum(y, 0.0)
  o_ref[...] = y


def _tc_layer(p, c, h, wl, wr, b, g, be, with_bn):
  return pl.pallas_call(
      functools.partial(_tc_layer_body, with_bn=with_bn),
      out_shape=jax.ShapeDtypeStruct((N, F), jnp.float32),
  )(p, c, h, wl, wr, b.reshape(1, F), g.reshape(1, F), be.reshape(1, F))


def kernel(x, edge_index, W_l0, W_r0, b0, gamma0, beta0,
           W_l1, W_r1, b1, gamma1, beta1, W_lo, W_ro, bo):
  src2d = edge_index[0].reshape(E // CH, CH)
  dst2d = edge_index[1].reshape(E // CH, CH)
  ones = jnp.ones((CH, CNTW), jnp.float32)
  zf = jnp.zeros((RPT, F), jnp.float32)
  zc = jnp.zeros((RPT, CNTW), jnp.float32)

  p0, c0 = _sc_agg_counts(x, src2d, dst2d, ones, zf, zc)
  h = _tc_layer(p0, c0, x, W_l0, W_r0, b0, gamma0, beta0, True)
  p1 = _sc_agg(h, src2d, dst2d, ones, zf, zc)
  h = _tc_layer(p1, c0, h, W_l1, W_r1, b1, gamma1, beta1, True)
  p2 = _sc_agg(h, src2d, dst2d, ones, zf, zc)
  out = _tc_layer(p2, c0, h, W_lo, W_ro, bo, bo, bo, False)
  return out


# speedup vs baseline: 4.0038x; 4.0038x over previous
import functools

import jax
import jax.numpy as jnp
from jax import lax
from jax.experimental import pallas as pl
from jax.experimental.pallas import tpu as pltpu
from jax.experimental.pallas import tpu_sc as plsc

N = 10000
E = 320000
F = 128
EPS = 1e-5

NC, NS = 2, 16
NW = NC * NS
EPW = E // NW
CH = 80
NCH = EPW // CH
NCHT = E // CH
PIECE = CH
NPIECES = N // PIECE
PPT = 8
CNTW = 16

_MESH = plsc.VectorSubcoreMesh(
    core_axis_name="c", subcore_axis_name="s", num_cores=NC, num_subcores=NS)


def _make_sc_pass(gather: bool):
  scratch = [
      pltpu.VMEM_SHARED((N, F), jnp.float32),
      pltpu.VMEM((CH,), jnp.int32),
      pltpu.VMEM((CH,), jnp.int32),
      pltpu.VMEM((CH, F), jnp.float32),
      pltpu.SemaphoreType.DMA,
      pltpu.VMEM((PIECE,), jnp.int32),
      pltpu.VMEM((PIECE,), jnp.int32),
  ]

  @functools.partial(
      pl.kernel, out_type=jax.ShapeDtypeStruct((NC * N, F), jnp.float32),
      mesh=_MESH, scratch_types=scratch)
  def sc_pass(h_hbm, src_hbm, dst_hbm, ones_hbm, zf_hbm, iota_hbm,
              out_hbm, agg_sh, src_v, dst_v, rows_v, sem, ibuf, obuf):
    cid = lax.axis_index("c")
    sid = lax.axis_index("s")
    wid = sid * NC + cid
    npieces = jnp.where(sid == NS - 1, NPIECES - (NS - 1) * PPT, PPT)

    pltpu.sync_copy(zf_hbm, rows_v)

    def zero_piece(p, carry):
      r = (sid * PPT + p) * PIECE
      pltpu.sync_copy(iota_hbm.at[pl.ds(r, PIECE)], ibuf)
      pltpu.sync_copy(rows_v, agg_sh.at[ibuf])
      return carry

    lax.fori_loop(0, npieces, zero_piece, 0)
    if not gather:
      pltpu.sync_copy(ones_hbm, rows_v)
    plsc.subcore_barrier()

    def chunk(j, carry):
      base = wid * EPW + j * CH
      pltpu.sync_copy(dst_hbm.at[pl.ds(base, CH)], dst_v)
      if gather:
        pltpu.sync_copy(src_hbm.at[pl.ds(base, CH)], src_v)
        pltpu.async_copy(h_hbm.at[src_v], rows_v, sem).wait()
      pltpu.sync_copy(rows_v, agg_sh.at[dst_v], add=True)
      return carry

    lax.fori_loop(0, NCH, chunk, 0)
    plsc.subcore_barrier()

    def wb_piece(p, carry):
      r = (sid * PPT + p) * PIECE
      pltpu.sync_copy(iota_hbm.at[pl.ds(r, PIECE)], ibuf)
      for k in range(PIECE // 16):
        obuf[pl.ds(k * 16, 16)] = ibuf[pl.ds(k * 16, 16)] + cid * N
      pltpu.async_copy(agg_sh.at[ibuf], rows_v, sem).wait()
      pltpu.sync_copy(rows_v, out_hbm.at[obuf])
      return carry

    lax.fori_loop(0, npieces, wb_piece, 0)

  return sc_pass


_sc_agg = _make_sc_pass(True)
_sc_counts = _make_sc_pass(False)


def _dot(a, b):
  return lax.dot_general(a, b, (((1,), (0,)), ((), ())),
                         precision=lax.Precision.HIGHEST,
                         preferred_element_type=jnp.float32)


RB = 400
NRB = N // RB


def _tc_lin_body(p_ref, c_ref, h_ref, wl_ref, wr_ref, b_ref, y_ref,
                 stats_ref, acc_ref, *, with_stats):
  i = pl.program_id(0)
  cnt = jnp.maximum(c_ref[0, :, 0:1] + c_ref[1, :, 0:1], 1.0)
  mean = (p_ref[0] + p_ref[1]) / cnt
  y = _dot(mean, wl_ref[...]) + _dot(h_ref[...], wr_ref[...]) + b_ref[...]
  y_ref[...] = y
  if with_stats:
    @pl.when(i == 0)
    def _():
      acc_ref[...] = jnp.zeros_like(acc_ref)
    acc_ref[0:1, :] += jnp.sum(y, axis=0, keepdims=True)
    acc_ref[1:2, :] += jnp.sum(y * y, axis=0, keepdims=True)

    @pl.when(i == NRB - 1)
    def _():
      stats_ref[...] = acc_ref[...]


def _tc_lin(p, c, h, wl, wr, b, with_stats):
  out_shape = [jax.ShapeDtypeStruct((N, F), jnp.float32),
               jax.ShapeDtypeStruct((8, F), jnp.float32)]
  grid = (NRB,)
  in_specs = [
      pl.BlockSpec((NC, RB, F), lambda i: (0, i, 0)),
      pl.BlockSpec((NC, RB, F), lambda i: (0, i, 0)),
      pl.BlockSpec((RB, F), lambda i: (i, 0)),
      pl.BlockSpec((F, F), lambda i: (0, 0)),
      pl.BlockSpec((F, F), lambda i: (0, 0)),
      pl.BlockSpec((1, F), lambda i: (0, 0)),
  ]
  out_specs = [
      pl.BlockSpec((RB, F), lambda i: (i, 0)),
      pl.BlockSpec((8, F), lambda i: (0, 0)),
  ]
  y, stats = pl.pallas_call(
      functools.partial(_tc_lin_body, with_stats=with_stats),
      grid=grid, in_specs=in_specs, out_specs=out_specs,
      out_shape=out_shape,
      scratch_shapes=[pltpu.VMEM((8, F), jnp.float32)],
  )(p, c, h, wl, wr, b.reshape(1, F))
  return y, stats


def _tc_bn_body(y_ref, stats_ref, g_ref, be_ref, o_ref):
  mu = stats_ref[0:1, :] * (1.0 / N)
  var = stats_ref[1:2, :] * (1.0 / N) - mu * mu
  y = y_ref[...]
  yn = (y - mu) * lax.rsqrt(var + EPS) * g_ref[...] + be_ref[...]
  o_ref[...] = jnp.maximum(yn, 0.0)


def _tc_bn(y, stats, g, be):
  return pl.pallas_call(
      _tc_bn_body,
      grid=(NRB,),
      in_specs=[
          pl.BlockSpec((RB, F), lambda i: (i, 0)),
          pl.BlockSpec((8, F), lambda i: (0, 0)),
          pl.BlockSpec((1, F), lambda i: (0, 0)),
          pl.BlockSpec((1, F), lambda i: (0, 0)),
      ],
      out_specs=pl.BlockSpec((RB, F), lambda i: (i, 0)),
      out_shape=jax.ShapeDtypeStruct((N, F), jnp.float32),
  )(y, stats, g.reshape(1, F), be.reshape(1, F))


def kernel(x, edge_index, W_l0, W_r0, b0, gamma0, beta0,
           W_l1, W_r1, b1, gamma1, beta1, W_lo, W_ro, bo):
  src1d = edge_index[0]
  dst1d = edge_index[1]
  ones = jnp.ones((CH, F), jnp.float32)
  zf = jnp.zeros((PIECE, F), jnp.float32)
  iota = jnp.arange(N, dtype=jnp.int32)

  c0 = _sc_counts(x, src1d, dst1d, ones, zf, iota).reshape(NC, N, F)
  p0 = _sc_agg(x, src1d, dst1d, ones, zf, iota).reshape(NC, N, F)
  y, s = _tc_lin(p0, c0, x, W_l0, W_r0, b0, True)
  h = _tc_bn(y, s, gamma0, beta0)
  p1 = _sc_agg(h, src1d, dst1d, ones, zf, iota)
  y, s = _tc_lin(p1.reshape(NC, N, F), c0, h, W_l1, W_r1, b1, True)
  h = _tc_bn(y, s, gamma1, beta1)
  p2 = _sc_agg(h, src1d, dst1d, ones, zf, iota)
  out, _ = _tc_lin(p2.reshape(NC, N, F), c0, h, W_lo, W_ro, bo, False)
  return out


# speedup vs baseline: 5.7856x; 1.4450x over previous
import functools

import jax
import jax.numpy as jnp
from jax import lax
from jax.experimental import pallas as pl
from jax.experimental.pallas import tpu as pltpu
from jax.experimental.pallas import tpu_sc as plsc

N = 10000
E = 320000
F = 128
EPS = 1e-5

NC, NS = 2, 16
NW = NC * NS
EPW = E // NW
CH = 80
NCH = EPW // CH
NCHT = E // CH
PIECE = CH
NPIECES = N // PIECE
PPT = 8
CNTW = 16

_MESH = plsc.VectorSubcoreMesh(
    core_axis_name="c", subcore_axis_name="s", num_cores=NC, num_subcores=NS)


def _make_sc_pass(gather: bool):
  scratch = [
      pltpu.VMEM_SHARED((N, F), jnp.float32),
      pltpu.VMEM((CH,), jnp.int32),
      pltpu.VMEM((CH,), jnp.int32),
      pltpu.VMEM((CH, F), jnp.float32),
      pltpu.SemaphoreType.DMA,
      pltpu.VMEM((PIECE,), jnp.int32),
      pltpu.VMEM((PIECE,), jnp.int32),
      pltpu.VMEM((CH,), jnp.int32),
      pltpu.VMEM((CH,), jnp.int32),
      pltpu.VMEM((CH, F), jnp.float32),
      pltpu.SemaphoreType.DMA,
  ]

  @functools.partial(
      pl.kernel, out_type=jax.ShapeDtypeStruct((NC * N, F), jnp.float32),
      mesh=_MESH, scratch_types=scratch)
  def sc_pass(h_hbm, src_hbm, dst_hbm, ones_hbm, zf_hbm, iota_hbm,
              out_hbm, agg_sh, src_v, dst_v, rows_v, sem, ibuf, obuf,
              src_v1, dst_v1, rows_v1, sem1):
    cid = lax.axis_index("c")
    sid = lax.axis_index("s")
    wid = sid * NC + cid
    npieces = jnp.where(sid == NS - 1, NPIECES - (NS - 1) * PPT, PPT)

    pltpu.sync_copy(zf_hbm, rows_v)

    def zero_piece(p, carry):
      r = (sid * PPT + p) * PIECE
      pltpu.sync_copy(iota_hbm.at[pl.ds(r, PIECE)], ibuf)
      pltpu.sync_copy(rows_v, agg_sh.at[ibuf])
      return carry

    lax.fori_loop(0, npieces, zero_piece, 0)
    if not gather:
      pltpu.sync_copy(ones_hbm, rows_v)
    plsc.subcore_barrier()

    e0 = wid * EPW
    if gather:
      pltpu.sync_copy(src_hbm.at[pl.ds(e0, CH)], src_v)
      pltpu.sync_copy(dst_hbm.at[pl.ds(e0, CH)], dst_v)
      pltpu.async_copy(h_hbm.at[src_v], rows_v, sem)

      def pair(j2, carry):
        a = 2 * j2
        pltpu.sync_copy(src_hbm.at[pl.ds(e0 + (a + 1) * CH, CH)], src_v1)
        pltpu.sync_copy(dst_hbm.at[pl.ds(e0 + (a + 1) * CH, CH)], dst_v1)
        pltpu.async_copy(h_hbm.at[src_v1], rows_v1, sem1)
        pltpu.make_async_copy(h_hbm.at[src_v], rows_v, sem).wait()
        pltpu.sync_copy(rows_v, agg_sh.at[dst_v], add=True)
        pltpu.sync_copy(src_hbm.at[pl.ds(e0 + (a + 2) * CH, CH)], src_v)
        pltpu.sync_copy(dst_hbm.at[pl.ds(e0 + (a + 2) * CH, CH)], dst_v)
        pltpu.async_copy(h_hbm.at[src_v], rows_v, sem)
        pltpu.make_async_copy(h_hbm.at[src_v1], rows_v1, sem1).wait()
        pltpu.sync_copy(rows_v1, agg_sh.at[dst_v1], add=True)
        return carry

      lax.fori_loop(0, (NCH - 1) // 2, pair, 0)
      pltpu.make_async_copy(h_hbm.at[src_v], rows_v, sem).wait()
      pltpu.sync_copy(rows_v, agg_sh.at[dst_v], add=True)
    else:

      def chunk(j, carry):
        base = e0 + j * CH
        pltpu.sync_copy(dst_hbm.at[pl.ds(base, CH)], dst_v)
        pltpu.sync_copy(rows_v, agg_sh.at[dst_v], add=True)
        return carry

      lax.fori_loop(0, NCH, chunk, 0)
    plsc.subcore_barrier()

    def wb_piece(p, carry):
      r = (sid * PPT + p) * PIECE
      pltpu.sync_copy(iota_hbm.at[pl.ds(r, PIECE)], ibuf)
      for k in range(PIECE // 16):
        obuf[pl.ds(k * 16, 16)] = ibuf[pl.ds(k * 16, 16)] + cid * N
      pltpu.async_copy(agg_sh.at[ibuf], rows_v, sem).wait()
      pltpu.sync_copy(rows_v, out_hbm.at[obuf])
      return carry

    lax.fori_loop(0, npieces, wb_piece, 0)

  return sc_pass


_sc_agg = _make_sc_pass(True)
_sc_counts = _make_sc_pass(False)


def _dot(a, b):
  return lax.dot_general(a, b, (((1,), (0,)), ((), ())),
                         precision=lax.Precision.HIGHEST,
                         preferred_element_type=jnp.float32)


RB = 400
NRB = N // RB


def _tc_lin_body(p_ref, c_ref, h_ref, wl_ref, wr_ref, b_ref, y_ref,
                 stats_ref, acc_ref, *, with_stats):
  i = pl.program_id(0)
  cnt = jnp.maximum(c_ref[0, :, 0:1] + c_ref[1, :, 0:1], 1.0)
  mean = (p_ref[0] + p_ref[1]) / cnt
  y = _dot(mean, wl_ref[...]) + _dot(h_ref[...], wr_ref[...]) + b_ref[...]
  y_ref[...] = y
  if with_stats:
    @pl.when(i == 0)
    def _():
      acc_ref[...] = jnp.zeros_like(acc_ref)
    acc_ref[0:1, :] += jnp.sum(y, axis=0, keepdims=True)
    acc_ref[1:2, :] += jnp.sum(y * y, axis=0, keepdims=True)

    @pl.when(i == NRB - 1)
    def _():
      stats_ref[...] = acc_ref[...]


def _tc_lin(p, c, h, wl, wr, b, with_stats):
  out_shape = [jax.ShapeDtypeStruct((N, F), jnp.float32),
               jax.ShapeDtypeStruct((8, F), jnp.float32)]
  grid = (NRB,)
  in_specs = [
      pl.BlockSpec((NC, RB, F), lambda i: (0, i, 0)),
      pl.BlockSpec((NC, RB, F), lambda i: (0, i, 0)),
      pl.BlockSpec((RB, F), lambda i: (i, 0)),
      pl.BlockSpec((F, F), lambda i: (0, 0)),
      pl.BlockSpec((F, F), lambda i: (0, 0)),
      pl.BlockSpec((1, F), lambda i: (0, 0)),
  ]
  out_specs = [
      pl.BlockSpec((RB, F), lambda i: (i, 0)),
      pl.BlockSpec((8, F), lambda i: (0, 0)),
  ]
  y, stats = pl.pallas_call(
      functools.partial(_tc_lin_body, with_stats=with_stats),
      grid=grid, in_specs=in_specs, out_specs=out_specs,
      out_shape=out_shape,
      scratch_shapes=[pltpu.VMEM((8, F), jnp.float32)],
  )(p, c, h, wl, wr, b.reshape(1, F))
  return y, stats


def _tc_bn_body(y_ref, stats_ref, g_ref, be_ref, o_ref):
  mu = stats_ref[0:1, :] * (1.0 / N)
  var = stats_ref[1:2, :] * (1.0 / N) - mu * mu
  y = y_ref[...]
  yn = (y - mu) * lax.rsqrt(var + EPS) * g_ref[...] + be_ref[...]
  o_ref[...] = jnp.maximum(yn, 0.0)


def _tc_bn(y, stats, g, be):
  return pl.pallas_call(
      _tc_bn_body,
      grid=(NRB,),
      in_specs=[
          pl.BlockSpec((RB, F), lambda i: (i, 0)),
          pl.BlockSpec((8, F), lambda i: (0, 0)),
          pl.BlockSpec((1, F), lambda i: (0, 0)),
          pl.BlockSpec((1, F), lambda i: (0, 0)),
      ],
      out_specs=pl.BlockSpec((RB, F), lambda i: (i, 0)),
      out_shape=jax.ShapeDtypeStruct((N, F), jnp.float32),
  )(y, stats, g.reshape(1, F), be.reshape(1, F))


def kernel(x, edge_index, W_l0, W_r0, b0, gamma0, beta0,
           W_l1, W_r1, b1, gamma1, beta1, W_lo, W_ro, bo):
  src1d = edge_index[0]
  dst1d = edge_index[1]
  ones = jnp.ones((CH, F), jnp.float32)
  zf = jnp.zeros((PIECE, F), jnp.float32)
  iota = jnp.arange(N, dtype=jnp.int32)

  c0 = _sc_counts(x, src1d, dst1d, ones, zf, iota).reshape(NC, N, F)
  p0 = _sc_agg(x, src1d, dst1d, ones, zf, iota).reshape(NC, N, F)
  y, s = _tc_lin(p0, c0, x, W_l0, W_r0, b0, True)
  h = _tc_bn(y, s, gamma0, beta0)
  p1 = _sc_agg(h, src1d, dst1d, ones, zf, iota)
  y, s = _tc_lin(p1.reshape(NC, N, F), c0, h, W_l1, W_r1, b1, True)
  h = _tc_bn(y, s, gamma1, beta1)
  p2 = _sc_agg(h, src1d, dst1d, ones, zf, iota)
  out, _ = _tc_lin(p2.reshape(NC, N, F), c0, h, W_lo, W_ro, bo, False)
  return out


# speedup vs baseline: 5.8383x; 1.0091x over previous
import functools

import jax
import jax.numpy as jnp
from jax import lax
from jax.experimental import pallas as pl
from jax.experimental.pallas import tpu as pltpu
from jax.experimental.pallas import tpu_sc as plsc

N = 10000
E = 320000
F = 128
EPS = 1e-5

NC, NS = 2, 16
NW = NC * NS
EPW = E // NW
CH = 80
NCH = EPW // CH
NCHT = E // CH
PIECE = CH
NPIECES = N // PIECE
PPT = 8
CNTW = 16

_MESH = plsc.VectorSubcoreMesh(
    core_axis_name="c", subcore_axis_name="s", num_cores=NC, num_subcores=NS)


def _make_sc_pass(gather: bool):
  scratch = [
      pltpu.VMEM_SHARED((N, F), jnp.float32),
      pltpu.VMEM((4 * CH,), jnp.int32),
      pltpu.VMEM((CH,), jnp.int32),
      pltpu.VMEM((CH, F), jnp.float32),
      pltpu.SemaphoreType.DMA,
      pltpu.VMEM((PIECE,), jnp.int32),
      pltpu.VMEM((PIECE,), jnp.int32),
      pltpu.VMEM((CH,), jnp.int32),
      pltpu.VMEM((CH, F), jnp.float32),
      pltpu.SemaphoreType.DMA,
  ]

  @functools.partial(
      pl.kernel, out_type=jax.ShapeDtypeStruct((NC * N, F), jnp.float32),
      mesh=_MESH, scratch_types=scratch)
  def sc_pass(h_hbm, src_hbm, dst_hbm, ones_hbm, zf_hbm, iota_hbm,
              out_hbm, agg_sh, srcblk_v, dst_v, rows_v, sem, ibuf, obuf,
              dst_v1, rows_v1, sem1):
    cid = lax.axis_index("c")
    sid = lax.axis_index("s")
    wid = sid * NC + cid
    npieces = jnp.where(sid == NS - 1, NPIECES - (NS - 1) * PPT, PPT)

    pltpu.sync_copy(zf_hbm, rows_v)

    def zero_piece(p, carry):
      r = (sid * PPT + p) * PIECE
      pltpu.sync_copy(iota_hbm.at[pl.ds(r, PIECE)], ibuf)
      pltpu.sync_copy(rows_v, agg_sh.at[ibuf])
      return carry

    lax.fori_loop(0, npieces, zero_piece, 0)
    if not gather:
      pltpu.sync_copy(ones_hbm, rows_v)
    plsc.subcore_barrier()

    e0 = wid * EPW
    if gather:
      def g_start(boff, rows, gsem):
        pltpu.async_copy(h_hbm.at[srcblk_v.at[pl.ds(boff, CH)]], rows, gsem)

      def g_wait(boff, rows, gsem):
        pltpu.make_async_copy(
            h_hbm.at[srcblk_v.at[pl.ds(boff, CH)]], rows, gsem).wait()

      def stage_dst(c, dbuf):
        pltpu.sync_copy(dst_hbm.at[pl.ds(e0 + c * CH, CH)], dbuf)

      pltpu.sync_copy(src_hbm.at[pl.ds(e0, 4 * CH)], srcblk_v)
      stage_dst(0, dst_v)
      g_start(0, rows_v, sem)

      def quad(q, carry):
        c0 = 4 * q
        g_start(CH, rows_v1, sem1)
        stage_dst(c0 + 1, dst_v1)
        g_wait(0, rows_v, sem)
        pltpu.sync_copy(rows_v, agg_sh.at[dst_v], add=True)
        g_start(2 * CH, rows_v, sem)
        stage_dst(c0 + 2, dst_v)
        g_wait(CH, rows_v1, sem1)
        pltpu.sync_copy(rows_v1, agg_sh.at[dst_v1], add=True)
        g_start(3 * CH, rows_v1, sem1)
        stage_dst(c0 + 3, dst_v1)
        g_wait(2 * CH, rows_v, sem)
        pltpu.sync_copy(rows_v, agg_sh.at[dst_v], add=True)
        g_wait(3 * CH, rows_v1, sem1)
        pltpu.sync_copy(rows_v1, agg_sh.at[dst_v1], add=True)
        nb = jnp.minimum(c0 + 4, NCH - 4)
        pltpu.sync_copy(src_hbm.at[pl.ds(e0 + nb * CH, 4 * CH)], srcblk_v)
        soff = (c0 + 4 - nb) * CH
        stage_dst(c0 + 4, dst_v)
        pltpu.async_copy(h_hbm.at[srcblk_v.at[pl.ds(soff, CH)]], rows_v, sem)
        return carry

      lax.fori_loop(0, NCH // 4, quad, 0)
      pltpu.make_async_copy(
          h_hbm.at[srcblk_v.at[pl.ds(3 * CH, CH)]], rows_v, sem).wait()
      pltpu.sync_copy(rows_v, agg_sh.at[dst_v], add=True)
    else:

      def chunk(j, carry):
        base = e0 + j * CH
        pltpu.sync_copy(dst_hbm.at[pl.ds(base, CH)], dst_v)
        pltpu.sync_copy(rows_v, agg_sh.at[dst_v], add=True)
        return carry

      lax.fori_loop(0, NCH, chunk, 0)
    plsc.subcore_barrier()

    def wb_piece(p, carry):
      r = (sid * PPT + p) * PIECE
      pltpu.sync_copy(iota_hbm.at[pl.ds(r, PIECE)], ibuf)
      for k in range(PIECE // 16):
        obuf[pl.ds(k * 16, 16)] = ibuf[pl.ds(k * 16, 16)] + cid * N
      pltpu.async_copy(agg_sh.at[ibuf], rows_v, sem).wait()
      pltpu.sync_copy(rows_v, out_hbm.at[obuf])
      return carry

    lax.fori_loop(0, npieces, wb_piece, 0)

  return sc_pass


_sc_agg = _make_sc_pass(True)
_sc_counts = _make_sc_pass(False)


def _dot(a, b):
  return lax.dot_general(a, b, (((1,), (0,)), ((), ())),
                         precision=lax.Precision.HIGHEST,
                         preferred_element_type=jnp.float32)


RB = 400
NRB = N // RB


def _tc_lin_body(p_ref, c_ref, h_ref, wl_ref, wr_ref, b_ref, y_ref,
                 stats_ref, acc_ref, *, with_stats):
  i = pl.program_id(0)
  cnt = jnp.maximum(c_ref[0, :, 0:1] + c_ref[1, :, 0:1], 1.0)
  mean = (p_ref[0] + p_ref[1]) / cnt
  y = _dot(mean, wl_ref[...]) + _dot(h_ref[...], wr_ref[...]) + b_ref[...]
  y_ref[...] = y
  if with_stats:
    @pl.when(i == 0)
    def _():
      acc_ref[...] = jnp.zeros_like(acc_ref)
    acc_ref[0:1, :] += jnp.sum(y, axis=0, keepdims=True)
    acc_ref[1:2, :] += jnp.sum(y * y, axis=0, keepdims=True)

    @pl.when(i == NRB - 1)
    def _():
      stats_ref[...] = acc_ref[...]


def _tc_lin(p, c, h, wl, wr, b, with_stats):
  out_shape = [jax.ShapeDtypeStruct((N, F), jnp.float32),
               jax.ShapeDtypeStruct((8, F), jnp.float32)]
  grid = (NRB,)
  in_specs = [
      pl.BlockSpec((NC, RB, F), lambda i: (0, i, 0)),
      pl.BlockSpec((NC, RB, F), lambda i: (0, i, 0)),
      pl.BlockSpec((RB, F), lambda i: (i, 0)),
      pl.BlockSpec((F, F), lambda i: (0, 0)),
      pl.BlockSpec((F, F), lambda i: (0, 0)),
      pl.BlockSpec((1, F), lambda i: (0, 0)),
  ]
  out_specs = [
      pl.BlockSpec((RB, F), lambda i: (i, 0)),
      pl.BlockSpec((8, F), lambda i: (0, 0)),
  ]
  y, stats = pl.pallas_call(
      functools.partial(_tc_lin_body, with_stats=with_stats),
      grid=grid, in_specs=in_specs, out_specs=out_specs,
      out_shape=out_shape,
      scratch_shapes=[pltpu.VMEM((8, F), jnp.float32)],
  )(p, c, h, wl, wr, b.reshape(1, F))
  return y, stats


def _tc_bn_body(y_ref, stats_ref, g_ref, be_ref, o_ref):
  mu = stats_ref[0:1, :] * (1.0 / N)
  var = stats_ref[1:2, :] * (1.0 / N) - mu * mu
  y = y_ref[...]
  yn = (y - mu) * lax.rsqrt(var + EPS) * g_ref[...] + be_ref[...]
  o_ref[...] = jnp.maximum(yn, 0.0)


def _tc_bn(y, stats, g, be):
  return pl.pallas_call(
      _tc_bn_body,
      grid=(NRB,),
      in_specs=[
          pl.BlockSpec((RB, F), lambda i: (i, 0)),
          pl.BlockSpec((8, F), lambda i: (0, 0)),
          pl.BlockSpec((1, F), lambda i: (0, 0)),
          pl.BlockSpec((1, F), lambda i: (0, 0)),
      ],
      out_specs=pl.BlockSpec((RB, F), lambda i: (i, 0)),
      out_shape=jax.ShapeDtypeStruct((N, F), jnp.float32),
  )(y, stats, g.reshape(1, F), be.reshape(1, F))


def kernel(x, edge_index, W_l0, W_r0, b0, gamma0, beta0,
           W_l1, W_r1, b1, gamma1, beta1, W_lo, W_ro, bo):
  src1d = edge_index[0]
  dst1d = edge_index[1]
  ones = jnp.ones((CH, F), jnp.float32)
  zf = jnp.zeros((PIECE, F), jnp.float32)
  iota = jnp.arange(N, dtype=jnp.int32)

  c0 = _sc_counts(x, src1d, dst1d, ones, zf, iota).reshape(NC, N, F)
  p0 = _sc_agg(x, src1d, dst1d, ones, zf, iota).reshape(NC, N, F)
  y, s = _tc_lin(p0, c0, x, W_l0, W_r0, b0, True)
  h = _tc_bn(y, s, gamma0, beta0)
  p1 = _sc_agg(h, src1d, dst1d, ones, zf, iota)
  y, s = _tc_lin(p1.reshape(NC, N, F), c0, h, W_l1, W_r1, b1, True)
  h = _tc_bn(y, s, gamma1, beta1)
  p2 = _sc_agg(h, src1d, dst1d, ones, zf, iota)
  out, _ = _tc_lin(p2.reshape(NC, N, F), c0, h, W_lo, W_ro, bo, False)
  return out


# speedup vs baseline: 6.2153x; 1.0646x over previous
import functools

import jax
import jax.numpy as jnp
from jax import lax
from jax.experimental import pallas as pl
from jax.experimental.pallas import tpu as pltpu
from jax.experimental.pallas import tpu_sc as plsc

N = 10000
E = 320000
F = 128
EPS = 1e-5

NC, NS = 2, 16
NW = NC * NS
EPW = E // NW
CH = 80
NCH = EPW // CH
NCHT = E // CH
PIECE = CH
NPIECES = N // PIECE
PPT = 8
CNTW = 16

_MESH = plsc.VectorSubcoreMesh(
    core_axis_name="c", subcore_axis_name="s", num_cores=NC, num_subcores=NS)


def _make_sc_pass(gather: bool):
  scratch = [
      pltpu.VMEM_SHARED((N, F), jnp.float32),
      pltpu.VMEM((4 * CH,), jnp.int32),
      pltpu.VMEM((CH,), jnp.int32),
      pltpu.VMEM((CH, F), jnp.float32),
      pltpu.SemaphoreType.DMA,
      pltpu.VMEM((PIECE,), jnp.int32),
      pltpu.VMEM((PIECE,), jnp.int32),
      pltpu.VMEM((CH,), jnp.int32),
      pltpu.VMEM((CH, F), jnp.float32),
      pltpu.SemaphoreType.DMA,
  ]

  @functools.partial(
      pl.kernel, out_type=jax.ShapeDtypeStruct((NC * N, F), jnp.float32),
      mesh=_MESH, scratch_types=scratch)
  def sc_pass(h_hbm, src_hbm, dst_hbm, ones_hbm, zf_hbm, iota_hbm,
              out_hbm, agg_sh, srcblk_v, dst_v, rows_v, sem, ibuf, obuf,
              dst_v1, rows_v1, sem1):
    cid = lax.axis_index("c")
    sid = lax.axis_index("s")
    wid = sid * NC + cid
    npieces = jnp.where(sid == NS - 1, NPIECES - (NS - 1) * PPT, PPT)

    pltpu.sync_copy(zf_hbm, rows_v)

    def zero_piece(p, carry):
      r = (sid * PPT + p) * PIECE
      pltpu.sync_copy(iota_hbm.at[pl.ds(r, PIECE)], ibuf)
      pltpu.sync_copy(rows_v, agg_sh.at[ibuf])
      return carry

    lax.fori_loop(0, npieces, zero_piece, 0)
    if not gather:
      pltpu.sync_copy(ones_hbm, rows_v)
    plsc.subcore_barrier()

    e0 = wid * EPW
    if gather:
      def g_start(boff, rows, gsem):
        pltpu.async_copy(h_hbm.at[srcblk_v.at[pl.ds(boff, CH)]], rows, gsem)

      def g_wait(boff, rows, gsem):
        pltpu.make_async_copy(
            h_hbm.at[srcblk_v.at[pl.ds(boff, CH)]], rows, gsem).wait()

      def stage_dst(c, dbuf):
        pltpu.sync_copy(dst_hbm.at[pl.ds(e0 + c * CH, CH)], dbuf)

      pltpu.sync_copy(src_hbm.at[pl.ds(e0, 4 * CH)], srcblk_v)
      stage_dst(0, dst_v)
      g_start(0, rows_v, sem)

      def quad(q, carry):
        c0 = 4 * q
        g_start(CH, rows_v1, sem1)
        stage_dst(c0 + 1, dst_v1)
        g_wait(0, rows_v, sem)
        pltpu.sync_copy(rows_v, agg_sh.at[dst_v], add=True)
        g_start(2 * CH, rows_v, sem)
        stage_dst(c0 + 2, dst_v)
        g_wait(CH, rows_v1, sem1)
        pltpu.sync_copy(rows_v1, agg_sh.at[dst_v1], add=True)
        g_start(3 * CH, rows_v1, sem1)
        stage_dst(c0 + 3, dst_v1)
        g_wait(2 * CH, rows_v, sem)
        pltpu.sync_copy(rows_v, agg_sh.at[dst_v], add=True)
        g_wait(3 * CH, rows_v1, sem1)
        pltpu.sync_copy(rows_v1, agg_sh.at[dst_v1], add=True)
        nb = jnp.minimum(c0 + 4, NCH - 4)
        pltpu.sync_copy(src_hbm.at[pl.ds(e0 + nb * CH, 4 * CH)], srcblk_v)
        soff = (c0 + 4 - nb) * CH
        stage_dst(c0 + 4, dst_v)
        pltpu.async_copy(h_hbm.at[srcblk_v.at[pl.ds(soff, CH)]], rows_v, sem)
        return carry

      lax.fori_loop(0, NCH // 4, quad, 0)
      pltpu.make_async_copy(
          h_hbm.at[srcblk_v.at[pl.ds(3 * CH, CH)]], rows_v, sem).wait()
      pltpu.sync_copy(rows_v, agg_sh.at[dst_v], add=True)
    else:
      pltpu.sync_copy(dst_hbm.at[pl.ds(e0, CH)], dst_v)
      pltpu.async_copy(rows_v, agg_sh.at[dst_v], sem, add=True)

      def cpair(j2, carry):
        a = 2 * j2
        pltpu.sync_copy(dst_hbm.at[pl.ds(e0 + (a + 1) * CH, CH)], dst_v1)
        pltpu.async_copy(rows_v, agg_sh.at[dst_v1], sem1, add=True)
        pltpu.make_async_copy(rows_v, agg_sh.at[dst_v], sem).wait()
        pltpu.sync_copy(dst_hbm.at[pl.ds(e0 + (a + 2) * CH, CH)], dst_v)
        pltpu.async_copy(rows_v, agg_sh.at[dst_v], sem, add=True)
        pltpu.make_async_copy(rows_v, agg_sh.at[dst_v1], sem1).wait()
        return carry

      lax.fori_loop(0, (NCH - 1) // 2, cpair, 0)
      pltpu.make_async_copy(rows_v, agg_sh.at[dst_v], sem).wait()
    plsc.subcore_barrier()

    def wb_piece(p, carry):
      r = (sid * PPT + p) * PIECE
      pltpu.sync_copy(iota_hbm.at[pl.ds(r, PIECE)], ibuf)
      for k in range(PIECE // 16):
        obuf[pl.ds(k * 16, 16)] = ibuf[pl.ds(k * 16, 16)] + cid * N
      pltpu.async_copy(agg_sh.at[ibuf], rows_v, sem).wait()
      pltpu.sync_copy(rows_v, out_hbm.at[obuf])
      return carry

    lax.fori_loop(0, npieces, wb_piece, 0)

  return sc_pass


_sc_agg = _make_sc_pass(True)
_sc_counts = _make_sc_pass(False)


def _dot(a, b):
  return lax.dot_general(a, b, (((1,), (0,)), ((), ())),
                         precision=lax.Precision.HIGHEST,
                         preferred_element_type=jnp.float32)


RB = 400
NRB = N // RB


def _tc_lin_body(p_ref, c_ref, h_ref, wl_ref, wr_ref, b_ref, y_ref,
                 stats_ref, acc_ref, *, with_stats):
  i = pl.program_id(0)
  cnt = jnp.maximum(c_ref[0, :, 0:1] + c_ref[1, :, 0:1], 1.0)
  mean = (p_ref[0] + p_ref[1]) / cnt
  y = _dot(mean, wl_ref[...]) + _dot(h_ref[...], wr_ref[...]) + b_ref[...]
  y_ref[...] = y
  if with_stats:
    @pl.when(i == 0)
    def _():
      acc_ref[...] = jnp.zeros_like(acc_ref)
    acc_ref[0:1, :] += jnp.sum(y, axis=0, keepdims=True)
    acc_ref[1:2, :] += jnp.sum(y * y, axis=0, keepdims=True)

    @pl.when(i == NRB - 1)
    def _():
      stats_ref[...] = acc_ref[...]


def _tc_lin(p, c, h, wl, wr, b, with_stats):
  out_shape = [jax.ShapeDtypeStruct((N, F), jnp.float32),
               jax.ShapeDtypeStruct((8, F), jnp.float32)]
  grid = (NRB,)
  in_specs = [
      pl.BlockSpec((NC, RB, F), lambda i: (0, i, 0)),
      pl.BlockSpec((NC, RB, F), lambda i: (0, i, 0)),
      pl.BlockSpec((RB, F), lambda i: (i, 0)),
      pl.BlockSpec((F, F), lambda i: (0, 0)),
      pl.BlockSpec((F, F), lambda i: (0, 0)),
      pl.BlockSpec((1, F), lambda i: (0, 0)),
  ]
  out_specs = [
      pl.BlockSpec((RB, F), lambda i: (i, 0)),
      pl.BlockSpec((8, F), lambda i: (0, 0)),
  ]
  y, stats = pl.pallas_call(
      functools.partial(_tc_lin_body, with_stats=with_stats),
      grid=grid, in_specs=in_specs, out_specs=out_specs,
      out_shape=out_shape,
      scratch_shapes=[pltpu.VMEM((8, F), jnp.float32)],
  )(p, c, h, wl, wr, b.reshape(1, F))
  return y, stats


def _tc_bn_body(y_ref, stats_ref, g_ref, be_ref, o_ref):
  mu = stats_ref[0:1, :] * (1.0 / N)
  var = stats_ref[1:2, :] * (1.0 / N) - mu * mu
  y = y_ref[...]
  yn = (y - mu) * lax.rsqrt(var + EPS) * g_ref[...] + be_ref[...]
  o_ref[...] = jnp.maximum(yn, 0.0)


def _tc_bn(y, stats, g, be):
  return pl.pallas_call(
      _tc_bn_body,
      grid=(NRB,),
      in_specs=[
          pl.BlockSpec((RB, F), lambda i: (i, 0)),
          pl.BlockSpec((8, F), lambda i: (0, 0)),
          pl.BlockSpec((1, F), lambda i: (0, 0)),
          pl.BlockSpec((1, F), lambda i: (0, 0)),
      ],
      out_specs=pl.BlockSpec((RB, F), lambda i: (i, 0)),
      out_shape=jax.ShapeDtypeStruct((N, F), jnp.float32),
  )(y, stats, g.reshape(1, F), be.reshape(1, F))


def kernel(x, edge_index, W_l0, W_r0, b0, gamma0, beta0,
           W_l1, W_r1, b1, gamma1, beta1, W_lo, W_ro, bo):
  src1d = edge_index[0]
  dst1d = edge_index[1]
  ones = jnp.ones((CH, F), jnp.float32)
  zf = jnp.zeros((PIECE, F), jnp.float32)
  iota = jnp.arange(N, dtype=jnp.int32)

  c0 = _sc_counts(x, src1d, dst1d, ones, zf, iota).reshape(NC, N, F)
  p0 = _sc_agg(x, src1d, dst1d, ones, zf, iota).reshape(NC, N, F)
  y, s = _tc_lin(p0, c0, x, W_l0, W_r0, b0, True)
  h = _tc_bn(y, s, gamma0, beta0)
  p1 = _sc_agg(h, src1d, dst1d, ones, zf, iota)
  y, s = _tc_lin(p1.reshape(NC, N, F), c0, h, W_l1, W_r1, b1, True)
  h = _tc_bn(y, s, gamma1, beta1)
  p2 = _sc_agg(h, src1d, dst1d, ones, zf, iota)
  out, _ = _tc_lin(p2.reshape(NC, N, F), c0, h, W_lo, W_ro, bo, False)
  return out


# speedup vs baseline: 6.2229x; 1.0012x over previous
import functools

import jax
import jax.numpy as jnp
from jax import lax
from jax.experimental import pallas as pl
from jax.experimental.pallas import tpu as pltpu
from jax.experimental.pallas import tpu_sc as plsc

N = 10000
E = 320000
F = 128
EPS = 1e-5

NC, NS = 2, 16
NW = NC * NS
EPW = E // NW
CH = 80
NCH = EPW // CH
PIECE = CH
NPIECES = N // PIECE
PPT = 8

_MESH = plsc.VectorSubcoreMesh(
    core_axis_name="c", subcore_axis_name="s", num_cores=NC, num_subcores=NS)


def _make_sc_pass(gather: bool):
  scratch = [
      pltpu.VMEM_SHARED((N, F), jnp.float32),
      pltpu.VMEM((4 * CH,), jnp.int32),
      pltpu.VMEM((CH,), jnp.int32),
      pltpu.VMEM((CH, F), jnp.float32),
      pltpu.SemaphoreType.DMA,
      pltpu.VMEM((PIECE,), jnp.int32),
      pltpu.VMEM((PIECE,), jnp.int32),
      pltpu.VMEM((CH,), jnp.int32),
      pltpu.VMEM((CH, F), jnp.float32),
      pltpu.SemaphoreType.DMA,
  ]

  @functools.partial(
      pl.kernel, out_type=jax.ShapeDtypeStruct((NC * N, F), jnp.float32),
      mesh=_MESH, scratch_types=scratch)
  def sc_pass(h_hbm, src_hbm, dst_hbm, ones_hbm, zf_hbm, iota_hbm,
              out_hbm, agg_sh, srcblk_v, dst_v, rows_v, sem, ibuf, obuf,
              dst_v1, rows_v1, sem1):
    cid = lax.axis_index("c")
    sid = lax.axis_index("s")
    wid = sid * NC + cid
    npieces = jnp.where(sid == NS - 1, NPIECES - (NS - 1) * PPT, PPT)

    pltpu.sync_copy(zf_hbm, rows_v)

    def zero_piece(p, carry):
      r = (sid * PPT + p) * PIECE
      pltpu.sync_copy(iota_hbm.at[pl.ds(r, PIECE)], ibuf)
      pltpu.sync_copy(rows_v, agg_sh.at[ibuf])
      return carry

    lax.fori_loop(0, npieces, zero_piece, 0)
    if not gather:
      pltpu.sync_copy(ones_hbm, rows_v)
    plsc.subcore_barrier()

    e0 = wid * EPW
    if gather:
      def g_start(boff, rows, gsem):
        pltpu.async_copy(h_hbm.at[srcblk_v.at[pl.ds(boff, CH)]], rows, gsem)

      def g_wait(boff, rows, gsem):
        pltpu.make_async_copy(
            h_hbm.at[srcblk_v.at[pl.ds(boff, CH)]], rows, gsem).wait()

      def stage_dst(c, dbuf):
        pltpu.sync_copy(dst_hbm.at[pl.ds(e0 + c * CH, CH)], dbuf)

      pltpu.sync_copy(src_hbm.at[pl.ds(e0, 4 * CH)], srcblk_v)
      stage_dst(0, dst_v)
      g_start(0, rows_v, sem)

      def quad(q, carry):
        c0 = 4 * q
        g_start(CH, rows_v1, sem1)
        stage_dst(c0 + 1, dst_v1)
        g_wait(0, rows_v, sem)
        pltpu.sync_copy(rows_v, agg_sh.at[dst_v], add=True)
        g_start(2 * CH, rows_v, sem)
        stage_dst(c0 + 2, dst_v)
        g_wait(CH, rows_v1, sem1)
        pltpu.sync_copy(rows_v1, agg_sh.at[dst_v1], add=True)
        g_start(3 * CH, rows_v1, sem1)
        stage_dst(c0 + 3, dst_v1)
        g_wait(2 * CH, rows_v, sem)
        pltpu.sync_copy(rows_v, agg_sh.at[dst_v], add=True)
        g_wait(3 * CH, rows_v1, sem1)
        pltpu.sync_copy(rows_v1, agg_sh.at[dst_v1], add=True)
        nb = jnp.minimum(c0 + 4, NCH - 4)
        pltpu.sync_copy(src_hbm.at[pl.ds(e0 + nb * CH, 4 * CH)], srcblk_v)
        soff = (c0 + 4 - nb) * CH
        stage_dst(c0 + 4, dst_v)
        pltpu.async_copy(h_hbm.at[srcblk_v.at[pl.ds(soff, CH)]], rows_v, sem)
        return carry

      lax.fori_loop(0, NCH // 4, quad, 0)
      pltpu.make_async_copy(
          h_hbm.at[srcblk_v.at[pl.ds(3 * CH, CH)]], rows_v, sem).wait()
      pltpu.sync_copy(rows_v, agg_sh.at[dst_v], add=True)
    else:
      pltpu.sync_copy(dst_hbm.at[pl.ds(e0, CH)], dst_v)
      pltpu.async_copy(rows_v, agg_sh.at[dst_v], sem, add=True)

      def cpair(j2, carry):
        a = 2 * j2
        pltpu.sync_copy(dst_hbm.at[pl.ds(e0 + (a + 1) * CH, CH)], dst_v1)
        pltpu.async_copy(rows_v, agg_sh.at[dst_v1], sem1, add=True)
        pltpu.make_async_copy(rows_v, agg_sh.at[dst_v], sem).wait()
        pltpu.sync_copy(dst_hbm.at[pl.ds(e0 + (a + 2) * CH, CH)], dst_v)
        pltpu.async_copy(rows_v, agg_sh.at[dst_v], sem, add=True)
        pltpu.make_async_copy(rows_v, agg_sh.at[dst_v1], sem1).wait()
        return carry

      lax.fori_loop(0, (NCH - 1) // 2, cpair, 0)
      pltpu.make_async_copy(rows_v, agg_sh.at[dst_v], sem).wait()
    plsc.subcore_barrier()

    def wb_piece(p, carry):
      r = (sid * PPT + p) * PIECE
      pltpu.sync_copy(iota_hbm.at[pl.ds(r, PIECE)], ibuf)
      for k in range(PIECE // 16):
        obuf[pl.ds(k * 16, 16)] = ibuf[pl.ds(k * 16, 16)] + cid * N
      pltpu.async_copy(agg_sh.at[ibuf], rows_v, sem).wait()
      pltpu.sync_copy(rows_v, out_hbm.at[obuf])
      return carry

    lax.fori_loop(0, npieces, wb_piece, 0)

  return sc_pass


_sc_agg = _make_sc_pass(True)
_sc_counts = _make_sc_pass(False)


def _dot(a, b):
  return lax.dot_general(a, b, (((1,), (0,)), ((), ())),
                         precision=lax.Precision.HIGHEST,
                         preferred_element_type=jnp.float32)


RB = 400
NRB = N // RB


def _tc_lin_body(p_ref, c_ref, h_ref, wl_ref, wr_ref, b_ref, y_ref,
                 stats_ref, acc_ref, *, with_stats):
  i = pl.program_id(0)
  cnt = jnp.maximum(c_ref[0, :, 0:1] + c_ref[1, :, 0:1], 1.0)
  mean = (p_ref[0] + p_ref[1]) / cnt
  y = _dot(mean, wl_ref[...]) + _dot(h_ref[...], wr_ref[...]) + b_ref[...]
  y_ref[...] = y
  if with_stats:
    @pl.when(i == 0)
    def _():
      acc_ref[...] = jnp.zeros_like(acc_ref)
    acc_ref[0:1, :] += jnp.sum(y, axis=0, keepdims=True)
    acc_ref[1:2, :] += jnp.sum(y * y, axis=0, keepdims=True)

    @pl.when(i == NRB - 1)
    def _():
      stats_ref[...] = acc_ref[...]


def _tc_lin(p, c, h, wl, wr, b, with_stats):
  out_shape = [jax.ShapeDtypeStruct((N, F), jnp.float32),
               jax.ShapeDtypeStruct((8, F), jnp.float32)]
  grid = (NRB,)
  in_specs = [
      pl.BlockSpec((NC, RB, F), lambda i: (0, i, 0)),
      pl.BlockSpec((NC, RB, F), lambda i: (0, i, 0)),
      pl.BlockSpec((RB, F), lambda i: (i, 0)),
      pl.BlockSpec((F, F), lambda i: (0, 0)),
      pl.BlockSpec((F, F), lambda i: (0, 0)),
      pl.BlockSpec((1, F), lambda i: (0, 0)),
  ]
  out_specs = [
      pl.BlockSpec((RB, F), lambda i: (i, 0)),
      pl.BlockSpec((8, F), lambda i: (0, 0)),
  ]
  y, stats = pl.pallas_call(
      functools.partial(_tc_lin_body, with_stats=with_stats),
      grid=grid, in_specs=in_specs, out_specs=out_specs,
      out_shape=out_shape,
      scratch_shapes=[pltpu.VMEM((8, F), jnp.float32)],
  )(p, c, h, wl, wr, b.reshape(1, F))
  return y, stats


def _tc_bn_body(y_ref, stats_ref, g_ref, be_ref, o_ref):
  mu = stats_ref[0:1, :] * (1.0 / N)
  var = stats_ref[1:2, :] * (1.0 / N) - mu * mu
  y = y_ref[...]
  yn = (y - mu) * lax.rsqrt(var + EPS) * g_ref[...] + be_ref[...]
  o_ref[...] = jnp.maximum(yn, 0.0)


def _tc_bn(y, stats, g, be):
  return pl.pallas_call(
      _tc_bn_body,
      grid=(NRB,),
      in_specs=[
          pl.BlockSpec((RB, F), lambda i: (i, 0)),
          pl.BlockSpec((8, F), lambda i: (0, 0)),
          pl.BlockSpec((1, F), lambda i: (0, 0)),
          pl.BlockSpec((1, F), lambda i: (0, 0)),
      ],
      out_specs=pl.BlockSpec((RB, F), lambda i: (i, 0)),
      out_shape=jax.ShapeDtypeStruct((N, F), jnp.float32),
  )(y, stats, g.reshape(1, F), be.reshape(1, F))


def kernel(x, edge_index, W_l0, W_r0, b0, gamma0, beta0,
           W_l1, W_r1, b1, gamma1, beta1, W_lo, W_ro, bo):
  src1d = edge_index[0]
  dst1d = edge_index[1]
  ones = jnp.ones((CH, F), jnp.float32)
  zf = jnp.zeros((PIECE, F), jnp.float32)
  iota = jnp.arange(N, dtype=jnp.int32)

  c0 = _sc_counts(x, src1d, dst1d, ones, zf, iota).reshape(NC, N, F)
  p0 = _sc_agg(x, src1d, dst1d, ones, zf, iota).reshape(NC, N, F)
  y, s = _tc_lin(p0, c0, x, W_l0, W_r0, b0, True)
  h = _tc_bn(y, s, gamma0, beta0)
  p1 = _sc_agg(h, src1d, dst1d, ones, zf, iota)
  y, s = _tc_lin(p1.reshape(NC, N, F), c0, h, W_l1, W_r1, b1, True)
  h = _tc_bn(y, s, gamma1, beta1)
  p2 = _sc_agg(h, src1d, dst1d, ones, zf, iota)
  out, _ = _tc_lin(p2.reshape(NC, N, F), c0, h, W_lo, W_ro, bo, False)
  return out
